# baseline (speedup 1.0000x reference)
import jax
import jax.numpy as jnp
from jax import lax
from jax.experimental import pallas as pl
from jax.experimental.pallas import tpu as pltpu
from jax.experimental.pallas import tpu_sc as plsc

B = 16384
F_CAT = 26
VOCAB = 100000
F_NUM = 13
D = 32
F_OUT = F_CAT + F_NUM

NC = 2
NS = 16
NW = NC * NS
L = 16

BC = 16
ROWS_PER_W = B // NW
CHUNKS = ROWS_PER_W // BC
NT = (BC * F_NUM) // L
GROWS = BC * 2 * L
CAT_STRIDE = BC * F_CAT + L


def _rsqrt_vec(x):
    i = plsc.bitcast(x, jnp.int32)
    y = plsc.bitcast(jnp.int32(0x5F3759DF) - (i >> 1), jnp.float32)
    xh = x * 0.5
    for _ in range(3):
        y = y * (1.5 - xh * y * y)
    return y


def _body(cat_hbm, num_hbm, tab_hbm, w_hbm, b_hbm, g_hbm, bt_hbm, out_hbm,
          cat_v, idx_v, num_v, stage_v, nout_v, par_v,
          gsem0, gsem1, osem0, osem1):
    wid = lax.axis_index("s") * NC + lax.axis_index("c")
    base = wid * ROWS_PER_W

    pltpu.sync_copy(w_hbm, par_v.at[0])
    pltpu.sync_copy(b_hbm, par_v.at[1])
    pltpu.sync_copy(g_hbm, par_v.at[2])
    pltpu.sync_copy(bt_hbm, par_v.at[3])
    w0 = par_v[0, pl.ds(0, L)]
    w1 = par_v[0, pl.ds(L, L)]
    bb0 = par_v[1, pl.ds(0, L)]
    bb1 = par_v[1, pl.ds(L, L)]
    g0 = par_v[2, pl.ds(0, L)]
    g1 = par_v[2, pl.ds(L, L)]
    bt0 = par_v[3, pl.ds(0, L)]
    bt1 = par_v[3, pl.ds(L, L)]

    def _lanesum(v):
        s = v[0]
        for i in range(1, L):
            s = s + v[i]
        return s

    sw = _lanesum(w0 + w1)
    sb = _lanesum(bb0 + bb1)
    sww = _lanesum(w0 * w0 + w1 * w1)
    swb = _lanesum(w0 * bb0 + w1 * bb1)
    sbb = _lanesum(bb0 * bb0 + bb1 * bb1)
    inv_d = jnp.float32(1.0 / D)
    mw = sw * inv_d
    mb = sb * inv_d
    a_m = sww * inv_d - mw * mw
    c_m = swb * inv_d - mw * mb
    v_m = sbb * inv_d - mb * mb
    c2 = c_m * 2.0
    veps = v_m + 1e-5
    a0 = (w0 - mw) * g0
    a1 = (w1 - mw) * g1
    cc0 = (bb0 - mb) * g0
    cc1 = (bb1 - mb) * g1

    iota = lax.iota(jnp.int32, L)
    off_lo = iota * VOCAB
    off_hi = lax.rem(iota + 16, jnp.int32(F_CAT)) * VOCAB

    cat_v[pl.ds(BC * F_CAT, L)] = iota * 0
    cat_v[pl.ds(CAT_STRIDE + BC * F_CAT, L)] = iota * 0

    gsems = [gsem0, gsem1]
    osems = [osem0, osem1]

    def drain_out(s):
        for b in range(BC):
            pltpu.make_async_copy(
                stage_v.at[s, pl.ds(b * 2 * L, F_CAT)],
                out_hbm.at[pl.ds(b * F_OUT, F_CAT)],
                osems[s],
            ).wait()
            pltpu.make_async_copy(
                nout_v.at[s, pl.ds(b * F_NUM, F_NUM)],
                out_hbm.at[pl.ds(b * F_OUT + F_CAT, F_NUM)],
                osems[s],
            ).wait()

    def prepare(c, s, drain):
        b0 = base + c * BC

        if drain:
            drain_out(s)

        cb = s * CAT_STRIDE
        pltpu.sync_copy(cat_hbm.at[pl.ds(b0 * F_CAT, BC * F_CAT)],
                        cat_v.at[pl.ds(cb, BC * F_CAT)])
        pltpu.sync_copy(num_hbm.at[pl.ds(b0 * F_NUM, BC * F_NUM)],
                        num_v.at[s])

        for b in range(BC):
            lo = plsc.load_gather(cat_v, [iota + (cb + b * F_CAT)])
            hi = plsc.load_gather(cat_v, [iota + (cb + b * F_CAT + L)])
            idx_v[s, pl.ds(b * 2 * L, L)] = lo + off_lo
            idx_v[s, pl.ds(b * 2 * L + L, L)] = hi + off_hi

        pltpu.async_copy(
            tab_hbm.at[idx_v.at[s]],
            stage_v.at[s],
            gsems[s],
        )

    def finish(c, s):
        b0 = base + c * BC

        for t in range(NT):
            x = num_v[s, pl.ds(t * L, L)]
            var = x * x * a_m + x * c2 + veps
            r = _rsqrt_vec(var)
            xr = x * r
            for l in range(L):
                p = t * L + l
                xs = xr[l]
                rs = r[l]
                nout_v[s, p, pl.ds(0, L)] = xs * a0 + (rs * cc0 + bt0)
                nout_v[s, p, pl.ds(L, L)] = xs * a1 + (rs * cc1 + bt1)

        pltpu.make_async_copy(
            tab_hbm.at[idx_v.at[s]],
            stage_v.at[s],
            gsems[s],
        ).wait()

        for b in range(BC):
            pltpu.async_copy(
                stage_v.at[s, pl.ds(b * 2 * L, F_CAT)],
                out_hbm.at[pl.ds((b0 + b) * F_OUT, F_CAT)],
                osems[s],
            )
            pltpu.async_copy(
                nout_v.at[s, pl.ds(b * F_NUM, F_NUM)],
                out_hbm.at[pl.ds((b0 + b) * F_OUT + F_CAT, F_NUM)],
                osems[s],
            )

    prepare(0, 0, drain=False)

    def g_body(g, _):
        c = 2 * g

        @pl.when(g >= 1)
        def _():
            drain_out(1)

        prepare(c + 1, 1, drain=False)
        finish(c, 0)

        @pl.when(g < CHUNKS // 2 - 1)
        def _():
            prepare(c + 2, 0, drain=True)

        finish(c + 1, 1)
        return 0

    lax.fori_loop(0, CHUNKS // 2, g_body, 0)

    for s in range(2):
        drain_out(s)


@jax.jit
def _run(cat_flat, num_flat, tab_flat, w, b, g, bt):
    mesh = plsc.VectorSubcoreMesh(
        core_axis_name="c", subcore_axis_name="s", num_cores=NC, num_subcores=NS
    )
    out = pl.kernel(
        _body,
        out_type=jax.ShapeDtypeStruct((B * F_OUT, D), jnp.float32),
        mesh=mesh,
        compiler_params=pltpu.CompilerParams(
            needs_layout_passes=False, use_tc_tiling_on_sc=False),
        scratch_types=[
            pltpu.VMEM((2 * CAT_STRIDE,), jnp.int32),
            pltpu.VMEM((2, GROWS), jnp.int32),
            pltpu.VMEM((2, BC * F_NUM), jnp.float32),
            pltpu.VMEM((2, GROWS, D), jnp.float32),
            pltpu.VMEM((2, BC * F_NUM, D), jnp.float32),
            pltpu.VMEM((4, D), jnp.float32),
            pltpu.SemaphoreType.DMA,
            pltpu.SemaphoreType.DMA,
            pltpu.SemaphoreType.DMA,
            pltpu.SemaphoreType.DMA,
        ],
    )(cat_flat, num_flat, tab_flat, w, b, g, bt)
    return out.reshape(B, F_OUT, D)


def kernel(categorical_features, numerical_features, tables, W_num, b_num,
           ln_gamma, ln_beta):
    cat_flat = categorical_features.astype(jnp.int32).reshape(-1)
    num_flat = numerical_features.reshape(-1)
    tab_flat = tables.reshape(F_CAT * VOCAB, D)
    return _run(cat_flat, num_flat, tab_flat, W_num, b_num, ln_gamma, ln_beta)
